# baseline (speedup 1.0000x reference)
import functools

import jax
import jax.numpy as jnp
from jax import lax
from jax.experimental import pallas as pl
from jax.experimental.pallas import tpu as pltpu
from jax.experimental.pallas import tpu_sc as plsc

_G = 16


def _make_gather(B, D, N):
  info = plsc.get_sparse_core_info()
  NC, NS = info.num_cores, info.num_subcores
  NW = NC * NS
  b_per_w = B // NW
  n_groups = b_per_w // _G
  mesh = plsc.VectorSubcoreMesh(core_axis_name="c", subcore_axis_name="s")

  @functools.partial(
      pl.kernel,
      mesh=mesh,
      out_type=jax.ShapeDtypeStruct((D, B), jnp.float32),
      scratch_types=[
          pltpu.VMEM((b_per_w,), jnp.int32),
          pltpu.VMEM((2, _G, D, 128), jnp.float32),
          pltpu.VMEM((D, b_per_w), jnp.float32),
          pltpu.SemaphoreType.DMA,
      ],
      compiler_params=pltpu.CompilerParams(needs_layout_passes=False),
  )
  def gather_kernel(idx_hbm, table_hbm, out_hbm, idx_v, tiles_v, cols_v, sem):
    wid = lax.axis_index("s") * NC + lax.axis_index("c")
    base = wid * b_per_w
    pltpu.sync_copy(idx_hbm.at[pl.ds(base, b_per_w)], idx_v)

    evec = lax.iota(jnp.int32, _G)

    def fire(g, buf):
      col0_vec = lax.shift_right_logical(idx_v[pl.ds(g * _G, _G)], 7) * 128
      for e in range(_G):
        col0 = pl.multiple_of(col0_vec[e], 128)
        for h in range(2):
          pltpu.async_copy(
              table_hbm.at[pl.ds(h * 8, 8), pl.ds(col0, 128)],
              tiles_v.at[buf, e, pl.ds(h * 8, 8)],
              sem,
          )

    def drain(buf):
      for e in range(_G):
        for h in range(2):
          pltpu.make_async_copy(
              table_hbm.at[pl.ds(0, 8), pl.ds(0, 128)],
              tiles_v.at[buf, e, pl.ds(h * 8, 8)],
              sem,
          ).wait()

    def extract(g, buf):
      jvec = idx_v[pl.ds(g * _G, _G)] & 127
      bvec = g * _G + evec
      for d in range(D):
        dvec = jnp.full((_G,), d, jnp.int32)
        val = plsc.load_gather(tiles_v.at[buf], [evec, dvec, jvec])
        plsc.store_scatter(cols_v, [dvec, bvec], val)

    fire(0, 0)

    def pair_body(p, carry):
      g0 = p * 2
      fire(g0 + 1, 1)
      drain(0)
      extract(g0, 0)

      @pl.when(g0 + 2 < n_groups)
      def _():
        fire(g0 + 2, 0)

      drain(1)
      extract(g0 + 1, 1)
      return carry

    lax.fori_loop(0, n_groups // 2, pair_body, 0)
    pltpu.sync_copy(cols_v, out_hbm.at[:, pl.ds(base, b_per_w)])

  return gather_kernel


def kernel(inputs, w):
  B = inputs.shape[0]
  N, D = w.shape
  idx = inputs.astype(jnp.int32)
  out_t = _make_gather(B, D, N)(idx, w.T)
  return out_t.T[:, :, None]


# speedup vs baseline: 1.0126x; 1.0126x over previous
import functools

import jax
import jax.numpy as jnp
from jax import lax
from jax.experimental import pallas as pl
from jax.experimental.pallas import tpu as pltpu
from jax.experimental.pallas import tpu_sc as plsc

_G = 16


def _make_gather(B, D, N):
  info = plsc.get_sparse_core_info()
  NC, NS = info.num_cores, info.num_subcores
  NW = NC * NS
  b_per_w = B // NW
  n_groups = b_per_w // _G
  mesh = plsc.VectorSubcoreMesh(core_axis_name="c", subcore_axis_name="s")

  @functools.partial(
      pl.kernel,
      mesh=mesh,
      out_type=jax.ShapeDtypeStruct((D, B), jnp.float32),
      scratch_types=[
          pltpu.VMEM((b_per_w,), jnp.int32),
          pltpu.VMEM((2, _G, D, 128), jnp.float32),
          pltpu.VMEM((D, b_per_w), jnp.float32),
          pltpu.SemaphoreType.DMA,
      ],
      compiler_params=pltpu.CompilerParams(needs_layout_passes=False),
  )
  def gather_kernel(idx_hbm, table_hbm, out_hbm, idx_v, tiles_v, cols_v, sem):
    wid = lax.axis_index("s") * NC + lax.axis_index("c")
    base = wid * b_per_w
    pltpu.sync_copy(idx_hbm.at[pl.ds(base, b_per_w)], idx_v)

    evec = lax.iota(jnp.int32, _G)

    def fire(g, buf):
      col0_vec = lax.shift_right_logical(idx_v[pl.ds(g * _G, _G)], 7) * 128
      for e in range(_G):
        col0 = pl.multiple_of(col0_vec[e], 128)
        pltpu.async_copy(
            table_hbm.at[:, pl.ds(col0, 128)], tiles_v.at[buf, e], sem
        )

    def drain(buf):
      for e in range(_G):
        pltpu.make_async_copy(
            table_hbm.at[:, pl.ds(0, 128)], tiles_v.at[buf, e], sem
        ).wait()

    def extract(g, buf):
      jvec = idx_v[pl.ds(g * _G, _G)] & 127
      bvec = g * _G + evec
      for d in range(D):
        dvec = jnp.full((_G,), d, jnp.int32)
        val = plsc.load_gather(tiles_v.at[buf], [evec, dvec, jvec])
        plsc.store_scatter(cols_v, [dvec, bvec], val)

    fire(0, 0)

    def pair_body(p, carry):
      g0 = p * 2
      fire(g0 + 1, 1)
      drain(0)
      extract(g0, 0)

      @pl.when(g0 + 2 < n_groups)
      def _():
        fire(g0 + 2, 0)

      drain(1)
      extract(g0 + 1, 1)
      return carry

    lax.fori_loop(0, n_groups // 2, pair_body, 0)
    pltpu.sync_copy(cols_v, out_hbm.at[:, pl.ds(base, b_per_w)])

  return gather_kernel


def kernel(inputs, w):
  B = inputs.shape[0]
  N, D = w.shape
  idx = inputs.astype(jnp.int32)
  out_t = _make_gather(B, D, N)(idx, w.T)
  return out_t.T[:, :, None]
